# Initial kernel scaffold; baseline (speedup 1.0000x reference)
import jax
import jax.numpy as jnp
from jax.experimental import pallas as pl


def kernel(x, edge_index, edge_attr, edge_weights, We0, be0, W10, b10, W20, b20, We1, be1, W11, b11, W21, b21, We2, be2, W12, b12, W22, b22, Wout, bout):
    raise NotImplementedError("write your pallas kernel here")


# speedup vs baseline: 2.8860x; 2.8860x over previous
import functools

import jax
import jax.numpy as jnp
from jax import lax
from jax.experimental import pallas as pl
from jax.experimental.pallas import tpu as pltpu
from jax.experimental.pallas import tpu_sc as plsc

N = 10000
E = 320000
H = 128
NT = 14
IN_DIM = 129
D0P = 144

NC = 2
NS = 16
NW = NC * NS
EPW = E // NW
SUB = 80
NCH = EPW // SUB
RPT = N // NS
L = 16


def _splat(v, i):
  return jnp.take_along_axis(
      v, jnp.full((L,), i, jnp.int32), axis=0, mode="promise_in_bounds"
  )


_sc_cache = {}


def _sc_agg(Dp):
  if Dp in _sc_cache:
    return _sc_cache[Dp]
  nj = Dp // L
  ng = SUB // L
  mesh = plsc.VectorSubcoreMesh(
      core_axis_name="c", subcore_axis_name="s", num_cores=NC, num_subcores=NS
  )

  @functools.partial(
      pl.kernel,
      out_type=jax.ShapeDtypeStruct((NC, N, Dp), jnp.float32),
      mesh=mesh,
      compiler_params=pltpu.CompilerParams(use_tc_tiling_on_sc=False),
      scratch_types=[
          pltpu.VMEM((SUB, Dp), jnp.float32),
          pltpu.VMEM((SUB,), jnp.int32),
          pltpu.VMEM((SUB,), jnp.int32),
          pltpu.VMEM((4, SUB), jnp.float32),
          pltpu.VMEM((4, Dp), jnp.float32),
          pltpu.VMEM_SHARED((N, Dp), jnp.float32),
      ],
  )
  def sc_agg(h_hbm, src_hbm, dst_hbm, ea_hbm, wep_hbm, out_hbm,
             rows_v, src_v, dst_v, ea_v, wep_v, agg_sh):
    c = lax.axis_index("c")
    s = lax.axis_index("s")
    wid = c * NS + s

    pltpu.sync_copy(wep_hbm, wep_v)
    wvec = [[wep_v[k, pl.ds(j * L, L)] for j in range(nj)] for k in range(4)]

    zero = jnp.zeros((L,), jnp.float32)

    def zrow(i, carry):
      for j in range(nj):
        rows_v[i, pl.ds(j * L, L)] = zero
      return carry

    lax.fori_loop(0, SUB, zrow, 0)
    for k in range(RPT // SUB):
      pltpu.sync_copy(rows_v, agg_sh.at[pl.ds(s * RPT + k * SUB, SUB)])
    rem = RPT % SUB
    pltpu.sync_copy(
        rows_v.at[pl.ds(0, rem)],
        agg_sh.at[pl.ds(s * RPT + (RPT - rem), rem)],
    )
    plsc.subcore_barrier()

    def chunk(it, carry):
      base = wid * EPW + it * SUB
      pltpu.sync_copy(src_hbm.at[pl.ds(base, SUB)], src_v)
      pltpu.sync_copy(dst_hbm.at[pl.ds(base, SUB)], dst_v)
      pltpu.sync_copy(ea_hbm.at[:, pl.ds(base, SUB)], ea_v)
      pltpu.sync_copy(h_hbm.at[src_v], rows_v)

      def group(g, carry2):
        gb = g * L
        w = ea_v[3, pl.ds(gb, L)]
        a0 = ea_v[0, pl.ds(gb, L)] * w
        a1 = ea_v[1, pl.ds(gb, L)] * w
        a2 = ea_v[2, pl.ds(gb, L)] * w
        for i in range(L):
          r = gb + i
          c0 = _splat(a0, i)
          c1 = _splat(a1, i)
          c2 = _splat(a2, i)
          for j in range(nj):
            js = pl.ds(j * L, L)
            e = c0 * wvec[0][j] + c1 * wvec[1][j] + c2 * wvec[2][j] + wvec[3][j]
            rows_v[r, js] = jnp.maximum(rows_v[r, js] + e, 0.0)
        return carry2

      lax.fori_loop(0, ng, group, 0)
      pltpu.sync_copy(rows_v, agg_sh.at[dst_v], add=True)
      return carry

    lax.fori_loop(0, NCH, chunk, 0)
    plsc.subcore_barrier()
    pltpu.sync_copy(
        agg_sh.at[pl.ds(s * RPT, RPT)], out_hbm.at[c, pl.ds(s * RPT, RPT)]
    )

  _sc_cache[Dp] = sc_agg
  return sc_agg


_B = 2000


def _mlp(h, a0, a1, W1, b1, W2, b2):
  Din = h.shape[1]

  def body(h_ref, a0_ref, a1_ref, W1_ref, b1_ref, W2_ref, b2_ref, o_ref):
    hin = h_ref[...] + a0_ref[...] + a1_ref[...]
    t = jnp.dot(hin, W1_ref[...], preferred_element_type=jnp.float32)
    t = jnp.maximum(t + b1_ref[...], 0.0)
    u = jnp.dot(t, W2_ref[...], preferred_element_type=jnp.float32)
    o_ref[...] = jnp.maximum(u + b2_ref[...], 0.0)

  return pl.pallas_call(
      body,
      grid=(N // _B,),
      in_specs=[
          pl.BlockSpec((_B, Din), lambda i: (i, 0)),
          pl.BlockSpec((_B, Din), lambda i: (i, 0)),
          pl.BlockSpec((_B, Din), lambda i: (i, 0)),
          pl.BlockSpec((Din, H), lambda i: (0, 0)),
          pl.BlockSpec((1, H), lambda i: (0, 0)),
          pl.BlockSpec((H, H), lambda i: (0, 0)),
          pl.BlockSpec((1, H), lambda i: (0, 0)),
      ],
      out_specs=pl.BlockSpec((_B, H), lambda i: (i, 0)),
      out_shape=jax.ShapeDtypeStruct((N, H), jnp.float32),
  )(h, a0, a1, W1, b1, W2, b2)


def _mlp_final(h, a0, a1, W1, b1, W2, b2, Wo, bo):
  Din = h.shape[1]

  def body(h_ref, a0_ref, a1_ref, W1_ref, b1_ref, W2_ref, b2_ref,
           Wo_ref, bo_ref, o_ref):
    hin = h_ref[...] + a0_ref[...] + a1_ref[...]
    t = jnp.dot(hin, W1_ref[...], preferred_element_type=jnp.float32)
    t = jnp.maximum(t + b1_ref[...], 0.0)
    u = jnp.dot(t, W2_ref[...], preferred_element_type=jnp.float32)
    u = jnp.maximum(u + b2_ref[...], 0.0)
    o_ref[...] = jnp.dot(u, Wo_ref[...],
                         preferred_element_type=jnp.float32) + bo_ref[...]

  return pl.pallas_call(
      body,
      grid=(N // _B,),
      in_specs=[
          pl.BlockSpec((_B, Din), lambda i: (i, 0)),
          pl.BlockSpec((_B, Din), lambda i: (i, 0)),
          pl.BlockSpec((_B, Din), lambda i: (i, 0)),
          pl.BlockSpec((Din, H), lambda i: (0, 0)),
          pl.BlockSpec((1, H), lambda i: (0, 0)),
          pl.BlockSpec((H, H), lambda i: (0, 0)),
          pl.BlockSpec((1, H), lambda i: (0, 0)),
          pl.BlockSpec((H, H), lambda i: (0, 0)),
          pl.BlockSpec((1, H), lambda i: (0, 0)),
      ],
      out_specs=pl.BlockSpec((_B, H), lambda i: (i, 0)),
      out_shape=jax.ShapeDtypeStruct((N, H), jnp.float32),
  )(h, a0, a1, W1, b1, W2, b2, Wo, bo)


def kernel(x, edge_index, edge_attr, edge_weights,
           We0, be0, W10, b10, W20, b20,
           We1, be1, W11, b11, W21, b21,
           We2, be2, W12, b12, W22, b22,
           Wout, bout):
  src = edge_index[0]
  dst = edge_index[1]
  eaT = jnp.concatenate([edge_attr.T, edge_weights[None, :]], axis=0)

  pad0 = D0P - IN_DIM
  h0 = jnp.pad(x, ((0, 0), (0, pad0)))
  wep0 = jnp.concatenate(
      [jnp.pad(We0, ((0, 0), (0, pad0))), jnp.pad(be0, (0, pad0))[None]], axis=0)
  agg = _sc_agg(D0P)(h0, src, dst, eaT, wep0)
  W10p = jnp.pad(W10, ((0, pad0), (0, 0)))
  h = _mlp(h0, agg[0], agg[1], W10p, b10[None], W20, b20[None])

  wep1 = jnp.concatenate([We1, be1[None]], axis=0)
  agg = _sc_agg(H)(h, src, dst, eaT, wep1)
  h = _mlp(h, agg[0], agg[1], W11, b11[None], W21, b21[None])

  wep2 = jnp.concatenate([We2, be2[None]], axis=0)
  agg = _sc_agg(H)(h, src, dst, eaT, wep2)
  Woutp = jnp.pad(Wout, ((0, 0), (0, H - NT)))
  boutp = jnp.pad(bout, (0, H - NT))
  out = _mlp_final(h, agg[0], agg[1], W12, b12[None], W22, b22[None],
                   Woutp, boutp[None])
  return out[:, :NT]


# speedup vs baseline: 6.7822x; 2.3501x over previous
import functools

import jax
import jax.numpy as jnp
from jax import lax
from jax.experimental import pallas as pl
from jax.experimental.pallas import tpu as pltpu
from jax.experimental.pallas import tpu_sc as plsc

N = 10000
E = 320000
H = 128
NT = 14
IN_DIM = 129
D0P = 144

NC = 2
NS = 16
NW = NC * NS
EPW = E // NW
SUB = 40
NCH = EPW // SUB
RPT = N // NS
L = 16


def _splat(v, i):
  return jnp.take_along_axis(
      v, jnp.full((L,), i, jnp.int32), axis=0, mode="promise_in_bounds"
  )


_sc_cache = {}


def _sc_agg(Dp):
  if Dp in _sc_cache:
    return _sc_cache[Dp]
  nj = Dp // L
  ng = SUB // L
  mesh = plsc.VectorSubcoreMesh(
      core_axis_name="c", subcore_axis_name="s", num_cores=NC, num_subcores=NS
  )

  @functools.partial(
      pl.kernel,
      out_type=jax.ShapeDtypeStruct((NC, N, Dp), jnp.float32),
      mesh=mesh,
      compiler_params=pltpu.CompilerParams(use_tc_tiling_on_sc=False),
      scratch_types=[
          pltpu.VMEM((SUB, Dp), jnp.float32),
          pltpu.VMEM((SUB, Dp), jnp.float32),
          pltpu.VMEM((SUB, Dp), jnp.float32),
          pltpu.VMEM((SUB, Dp), jnp.float32),
          pltpu.VMEM((SUB,), jnp.int32),
          pltpu.VMEM((SUB,), jnp.int32),
          pltpu.VMEM((SUB,), jnp.int32),
          pltpu.VMEM((SUB,), jnp.int32),
          pltpu.VMEM((4, SUB), jnp.float32),
          pltpu.VMEM((4, SUB), jnp.float32),
          pltpu.VMEM((4, Dp), jnp.float32),
          pltpu.VMEM_SHARED((N, Dp), jnp.float32),
          pltpu.SemaphoreType.DMA,
          pltpu.SemaphoreType.DMA,
          pltpu.SemaphoreType.DMA,
          pltpu.SemaphoreType.DMA,
          pltpu.SemaphoreType.DMA,
          pltpu.SemaphoreType.DMA,
          pltpu.SemaphoreType.DMA,
          pltpu.SemaphoreType.DMA,
          pltpu.SemaphoreType.DMA,
          pltpu.SemaphoreType.DMA,
      ],
  )
  def sc_agg(h_hbm, src_hbm, dst_hbm, ea_hbm, wep_hbm, out_hbm,
             rows0, rows1, msg0, msg1, src0, src1, dst0, dst1, ea0, ea1, wep_v,
             agg_sh, gsem0, gsem1, ssem0, ssem1, rsem0, rsem1,
             dsem0, dsem1, esem0, esem1):
    c = lax.axis_index("c")
    s = lax.axis_index("s")
    wid = c * NS + s
    rows = (rows0, rows1)
    msg = (msg0, msg1)
    src_v = (src0, src1)
    dst_v = (dst0, dst1)
    ea_v = (ea0, ea1)
    gsem = (gsem0, gsem1)
    ssem = (ssem0, ssem1)
    rsem = (rsem0, rsem1)
    dsem = (dsem0, dsem1)
    esem = (esem0, esem1)

    pltpu.sync_copy(wep_hbm, wep_v)

    zero = jnp.zeros((L,), jnp.float32)

    def zrow(i, carry):
      for j in range(nj):
        msg0[i, pl.ds(j * L, L)] = zero
      return carry

    lax.fori_loop(0, SUB, zrow, 0)
    for k in range(RPT // SUB):
      pltpu.sync_copy(msg0, agg_sh.at[pl.ds(s * RPT + k * SUB, SUB)])
    rem = RPT % SUB
    pltpu.sync_copy(
        msg0.at[pl.ds(0, rem)],
        agg_sh.at[pl.ds(s * RPT + (RPT - rem), rem)],
    )
    plsc.subcore_barrier()

    def start_gather(p):
      pltpu.async_copy(h_hbm.at[src_v[p]], rows[p], gsem[p])

    def wait_gather(p):
      pltpu.make_async_copy(h_hbm.at[src_v[p]], rows[p], gsem[p]).wait()

    def start_src(i, p):
      pltpu.async_copy(src_hbm.at[i], src_v[p], rsem[p])

    def wait_src(i, p):
      pltpu.make_async_copy(src_hbm.at[i], src_v[p], rsem[p]).wait()

    def start_scatter(i, p):
      del i
      pltpu.async_copy(msg[p], agg_sh.at[dst_v[p]], ssem[p], add=True)

    def wait_scatter(p):
      pltpu.make_async_copy(msg[p], agg_sh.at[dst_v[p]], ssem[p]).wait()

    def start_dst(i, p):
      pltpu.async_copy(dst_hbm.at[i], dst_v[p], dsem[p])

    def wait_dst(i, p):
      pltpu.make_async_copy(dst_hbm.at[i], dst_v[p], dsem[p]).wait()

    def start_ea(i, p):
      pltpu.async_copy(
          ea_hbm.at[:, pl.ds(wid * EPW + i * SUB, SUB)], ea_v[p], esem[p]
      )

    def wait_ea(i, p):
      pltpu.make_async_copy(
          ea_hbm.at[:, pl.ds(wid * EPW + i * SUB, SUB)], ea_v[p], esem[p]
      ).wait()

    def process(i, p, steady, issue_next):
      wait_gather(p)
      pl.when(issue_next)(lambda: start_src(wid * NCH + i + 2, p))
      pl.when(steady)(lambda: wait_scatter(p))
      start_dst(wid * NCH + i, p)
      pl.when(steady)(lambda: wait_ea(i, p))
      wvec = [[wep_v[k, pl.ds(j * L, L)] for j in range(nj)] for k in range(4)]

      def group(g, carry2):
        gb = g * L
        w = ea_v[p][3, pl.ds(gb, L)]
        a0 = ea_v[p][0, pl.ds(gb, L)] * w
        a1 = ea_v[p][1, pl.ds(gb, L)] * w
        a2 = ea_v[p][2, pl.ds(gb, L)] * w
        for ii in range(L):
          r = gb + ii
          c0 = _splat(a0, ii)
          c1 = _splat(a1, ii)
          c2 = _splat(a2, ii)
          for j in range(nj):
            js = pl.ds(j * L, L)
            e = c0 * wvec[0][j] + c1 * wvec[1][j] + c2 * wvec[2][j] + wvec[3][j]
            msg[p][r, js] = jnp.maximum(rows[p][r, js] + e, 0.0)
        return carry2

      lax.fori_loop(0, ng, group, 0)

      def prefetch():
        start_ea(i + 2, p)
        wait_src(wid * NCH + i + 2, p)
        start_gather(p)

      pl.when(issue_next)(prefetch)
      wait_dst(wid * NCH + i, p)
      start_scatter(i, p)

    assert NCH % 2 == 0
    pltpu.sync_copy(src_hbm.at[wid * NCH], src0)
    start_gather(0)
    pltpu.sync_copy(src_hbm.at[wid * NCH + 1], src1)
    start_gather(1)
    pltpu.sync_copy(ea_hbm.at[:, pl.ds(wid * EPW, SUB)], ea0)
    pltpu.sync_copy(ea_hbm.at[:, pl.ds(wid * EPW + SUB, SUB)], ea1)

    def pair(t, carry):
      steady = t >= 1
      issue_next = t < NCH // 2 - 1
      process(2 * t, 0, steady, issue_next)
      process(2 * t + 1, 1, steady, issue_next)
      return carry

    lax.fori_loop(0, NCH // 2, pair, 0)
    wait_scatter(0)
    wait_scatter(1)

    plsc.subcore_barrier()
    pltpu.sync_copy(
        agg_sh.at[pl.ds(s * RPT, RPT)], out_hbm.at[c, pl.ds(s * RPT, RPT)]
    )

  _sc_cache[Dp] = sc_agg
  return sc_agg


_B = 2000


def _mlp(h, a0, a1, W1, b1, W2, b2):
  Din = h.shape[1]

  def body(h_ref, a0_ref, a1_ref, W1_ref, b1_ref, W2_ref, b2_ref, o_ref):
    hin = h_ref[...] + a0_ref[...] + a1_ref[...]
    t = jnp.dot(hin, W1_ref[...], preferred_element_type=jnp.float32)
    t = jnp.maximum(t + b1_ref[...], 0.0)
    u = jnp.dot(t, W2_ref[...], preferred_element_type=jnp.float32)
    o_ref[...] = jnp.maximum(u + b2_ref[...], 0.0)

  return pl.pallas_call(
      body,
      grid=(N // _B,),
      in_specs=[
          pl.BlockSpec((_B, Din), lambda i: (i, 0)),
          pl.BlockSpec((_B, Din), lambda i: (i, 0)),
          pl.BlockSpec((_B, Din), lambda i: (i, 0)),
          pl.BlockSpec((Din, H), lambda i: (0, 0)),
          pl.BlockSpec((1, H), lambda i: (0, 0)),
          pl.BlockSpec((H, H), lambda i: (0, 0)),
          pl.BlockSpec((1, H), lambda i: (0, 0)),
      ],
      out_specs=pl.BlockSpec((_B, H), lambda i: (i, 0)),
      out_shape=jax.ShapeDtypeStruct((N, H), jnp.float32),
  )(h, a0, a1, W1, b1, W2, b2)


def _mlp_final(h, a0, a1, W1, b1, W2, b2, Wo, bo):
  Din = h.shape[1]

  def body(h_ref, a0_ref, a1_ref, W1_ref, b1_ref, W2_ref, b2_ref,
           Wo_ref, bo_ref, o_ref):
    hin = h_ref[...] + a0_ref[...] + a1_ref[...]
    t = jnp.dot(hin, W1_ref[...], preferred_element_type=jnp.float32)
    t = jnp.maximum(t + b1_ref[...], 0.0)
    u = jnp.dot(t, W2_ref[...], preferred_element_type=jnp.float32)
    u = jnp.maximum(u + b2_ref[...], 0.0)
    o_ref[...] = jnp.dot(u, Wo_ref[...],
                         preferred_element_type=jnp.float32) + bo_ref[...]

  return pl.pallas_call(
      body,
      grid=(N // _B,),
      in_specs=[
          pl.BlockSpec((_B, Din), lambda i: (i, 0)),
          pl.BlockSpec((_B, Din), lambda i: (i, 0)),
          pl.BlockSpec((_B, Din), lambda i: (i, 0)),
          pl.BlockSpec((Din, H), lambda i: (0, 0)),
          pl.BlockSpec((1, H), lambda i: (0, 0)),
          pl.BlockSpec((H, H), lambda i: (0, 0)),
          pl.BlockSpec((1, H), lambda i: (0, 0)),
          pl.BlockSpec((H, H), lambda i: (0, 0)),
          pl.BlockSpec((1, H), lambda i: (0, 0)),
      ],
      out_specs=pl.BlockSpec((_B, H), lambda i: (i, 0)),
      out_shape=jax.ShapeDtypeStruct((N, H), jnp.float32),
  )(h, a0, a1, W1, b1, W2, b2, Wo, bo)


def kernel(x, edge_index, edge_attr, edge_weights,
           We0, be0, W10, b10, W20, b20,
           We1, be1, W11, b11, W21, b21,
           We2, be2, W12, b12, W22, b22,
           Wout, bout):
  src = edge_index[0].reshape(E // SUB, SUB)
  dst = edge_index[1].reshape(E // SUB, SUB)
  eaT = jnp.concatenate([edge_attr.T, edge_weights[None, :]], axis=0)

  pad0 = D0P - IN_DIM
  h0 = jnp.pad(x, ((0, 0), (0, pad0)))
  wep0 = jnp.concatenate(
      [jnp.pad(We0, ((0, 0), (0, pad0))), jnp.pad(be0, (0, pad0))[None]], axis=0)
  agg = _sc_agg(D0P)(h0, src, dst, eaT, wep0)
  W10p = jnp.pad(W10, ((0, pad0), (0, 0)))
  h = _mlp(h0, agg[0], agg[1], W10p, b10[None], W20, b20[None])

  wep1 = jnp.concatenate([We1, be1[None]], axis=0)
  agg = _sc_agg(H)(h, src, dst, eaT, wep1)
  h = _mlp(h, agg[0], agg[1], W11, b11[None], W21, b21[None])

  wep2 = jnp.concatenate([We2, be2[None]], axis=0)
  agg = _sc_agg(H)(h, src, dst, eaT, wep2)
  Woutp = jnp.pad(Wout, ((0, 0), (0, H - NT)))
  boutp = jnp.pad(bout, (0, H - NT))
  out = _mlp_final(h, agg[0], agg[1], W12, b12[None], W22, b22[None],
                   Woutp, boutp[None])
  return out[:, :NT]
